# baseline (speedup 1.0000x reference)
import functools

import jax
import jax.numpy as jnp
from jax import lax
from jax.experimental import pallas as pl
from jax.experimental.pallas import tpu as pltpu
from jax.experimental.pallas import tpu_sc as plsc

D = 128
NC = 2
NS = 16
NW = NC * NS
CHUNK = 128
BR = 5000


@functools.lru_cache(maxsize=None)
def _make_segsum(n_acc, nchunk, n_extra):
    assert nchunk % 2 == 0 and nchunk >= 4 and 0 <= n_extra <= NW
    rows_per_tile = n_acc // NS
    assert rows_per_tile * NS == n_acc and rows_per_tile % 128 == 0
    mesh = plsc.VectorSubcoreMesh(core_axis_name="c", subcore_axis_name="s")

    @functools.partial(
        pl.kernel,
        out_type=jax.ShapeDtypeStruct((NC, n_acc, D), jnp.float32),
        mesh=mesh,
        scratch_types=[
            pltpu.VMEM_SHARED((n_acc, D), jnp.float32),
            [pltpu.VMEM((2, CHUNK), jnp.int32) for _ in range(2)],
            [pltpu.VMEM((CHUNK, D), jnp.float32) for _ in range(2)],
            pltpu.VMEM((2, CHUNK), jnp.int32),
            [pltpu.SemaphoreType.DMA for _ in range(2)],
            [pltpu.SemaphoreType.DMA for _ in range(2)],
            pltpu.SemaphoreType.DMA,
            pltpu.SemaphoreType.DMA,
        ],
    )
    def segsum(edge_hbm, table_hbm, zeros_hbm, out_hbm,
               acc, eidx, bufs, eidx_t,
               isems, gsems, isem_t, gsem_t):
        c = lax.axis_index("c")
        s = lax.axis_index("s")
        w = s * NC + c
        base = w * (nchunk * CHUNK)

        def fire_idx(j, b):
            off = pl.multiple_of(base + j * CHUNK, CHUNK)
            pltpu.async_copy(
                edge_hbm.at[:, pl.ds(off, CHUNK)], eidx[b], isems[b])

        def wait_idx(j, b):
            off = pl.multiple_of(base + j * CHUNK, CHUNK)
            pltpu.make_async_copy(
                edge_hbm.at[:, pl.ds(off, CHUNK)], eidx[b], isems[b]).wait()

        def fire_gather(b):
            pltpu.async_copy(table_hbm.at[eidx[b].at[0]], bufs[b], gsems[b])

        def step(j, b, bn, fire_next_gather, fire_next_idx):
            if fire_next_gather:
                wait_idx(j + 1, bn)
                fire_gather(bn)
            pltpu.make_async_copy(
                table_hbm.at[eidx[b].at[0]], bufs[b], gsems[b]).wait()
            pltpu.sync_copy(bufs[b], acc.at[eidx[b].at[1]], add=True)
            if fire_next_idx:
                fire_idx(j + 2, b)

        off_t = pl.multiple_of((nchunk * NW + w) * CHUNK, CHUNK)

        fire_idx(0, 0)
        fire_idx(1, 1)
        if n_extra:
            @pl.when(w < n_extra)
            def _():
                pltpu.async_copy(
                    edge_hbm.at[:, pl.ds(off_t, CHUNK)], eidx_t, isem_t)
        wait_idx(0, 0)
        fire_gather(0)
        if n_extra:
            @pl.when(w < n_extra)
            def _():
                pltpu.make_async_copy(
                    edge_hbm.at[:, pl.ds(off_t, CHUNK)], eidx_t, isem_t).wait()
                pltpu.async_copy(table_hbm.at[eidx_t.at[0]], bufs[1], gsem_t)

        r0 = s * rows_per_tile
        pltpu.sync_copy(zeros_hbm.at[pl.ds(r0, rows_per_tile)],
                        acc.at[pl.ds(r0, rows_per_tile)])
        plsc.subcore_barrier()

        if n_extra:
            @pl.when(w < n_extra)
            def _():
                pltpu.make_async_copy(
                    table_hbm.at[eidx_t.at[0]], bufs[1], gsem_t).wait()
                pltpu.sync_copy(bufs[1], acc.at[eidx_t.at[1]], add=True)

        @pl.loop(0, nchunk - 2, step=2)
        def _(g):
            step(g, 0, 1, True, True)
            step(g + 1, 1, 0, True, True)

        step(nchunk - 2, 0, 1, True, False)
        step(nchunk - 1, 1, 0, False, False)

        plsc.subcore_barrier()
        pltpu.sync_copy(acc.at[pl.ds(r0, rows_per_tile)],
                        out_hbm.at[c].at[pl.ds(r0, rows_per_tile)])

    return segsum


def _tc1_body(p_ref, x_ref, wr_ref, wo_ref, b_ref, h_ref):
    agg = p_ref[0] + p_ref[1]
    h = (jnp.dot(agg, wr_ref[...], preferred_element_type=jnp.float32)
         + jnp.dot(x_ref[...], wo_ref[...], preferred_element_type=jnp.float32)
         + b_ref[...])
    h_ref[...] = jnp.maximum(h, 0.0)


def _tc2_body(p_ref, h_ref, wr_ref, wo_ref, b_ref, o_ref):
    agg = p_ref[0] + p_ref[1]
    o = (jnp.dot(agg, wr_ref[...], preferred_element_type=jnp.float32)
         + jnp.dot(h_ref[...], wo_ref[...], preferred_element_type=jnp.float32)
         + b_ref[...])
    o = o - jnp.max(o, axis=1, keepdims=True)
    o_ref[...] = o - jnp.log(jnp.sum(jnp.exp(o), axis=1, keepdims=True))


def _tc_layer(body, partials, dense_in, w_rel, w_root, b, n):
    grid = (n // BR,)
    return pl.pallas_call(
        body,
        grid=grid,
        in_specs=[
            pl.BlockSpec((NC, BR, D), lambda i: (0, i, 0)),
            pl.BlockSpec((BR, D), lambda i: (i, 0)),
            pl.BlockSpec((D, D), lambda i: (0, 0)),
            pl.BlockSpec((D, D), lambda i: (0, 0)),
            pl.BlockSpec((1, D), lambda i: (0, 0)),
        ],
        out_specs=pl.BlockSpec((BR, D), lambda i: (i, 0)),
        out_shape=jax.ShapeDtypeStruct((n, D), jnp.float32),
    )(partials, dense_in, w_rel, w_root, b.reshape(1, D))


def kernel(x, edge_index, W1_rel, W1_root, b1, W2_rel, W2_root, b2):
    n = x.shape[0]
    e = edge_index.shape[1]
    assert e % CHUNK == 0
    tot = e // CHUNK
    nchunk = (tot // NW) & ~1
    n_extra = tot - nchunk * NW
    assert n_extra <= NW

    n_acc = -(-n // (NS * 128)) * (NS * 128)
    zeros = jnp.zeros((n_acc, D), jnp.float32)
    segsum = _make_segsum(n_acc, nchunk, n_extra)
    p1 = segsum(edge_index, x, zeros)
    h = _tc_layer(_tc1_body, p1, x, W1_rel, W1_root, b1, n)
    p2 = segsum(edge_index, h, zeros)
    return _tc_layer(_tc2_body, p2, h, W2_rel, W2_root, b2, n)
